# baseline (speedup 1.0000x reference)
import functools

import jax
import jax.numpy as jnp
from jax import lax
from jax.experimental import pallas as pl
from jax.experimental.pallas import tpu as pltpu
from jax.experimental.pallas import tpu_sc as plsc

_N = 64
_INV_TEMP = 1.0 / (1.0 + 1e-10)
_EPS = 1e-10
_L = 16
_NV = _N // _L


def _router_body(w_hbm, out_hbm, w_v, out_v):
    wid = lax.axis_index("s") * 2 + lax.axis_index("c")

    @pl.when(wid == 0)
    def _():
        pltpu.sync_copy(w_hbm, w_v)
        vs = [w_v[pl.ds(i * _L, _L)] for i in range(_NV)]

        m = jnp.max(vs[0])
        for v in vs[1:]:
            m = jnp.maximum(m, jnp.max(v))

        s = jnp.float32(0.0)
        for v in vs:
            s = s + jnp.sum(jnp.exp((v - m) * jnp.float32(_INV_TEMP)))

        iota = lax.iota(jnp.int32, _L)
        first = jnp.int32(_N)
        for i, v in enumerate(vs):
            cand = jnp.where(v == m, iota + jnp.int32(i * _L), jnp.int32(_N))
            first = jnp.minimum(first, jnp.min(cand))

        ones = jnp.ones((_L,), jnp.float32)
        denom = jnp.full((_L,), 1.0, jnp.float32) + jnp.float32(_EPS) * s
        val = ones / denom
        zero = jnp.zeros((_L,), jnp.float32)
        for i in range(_NV):
            idxs = iota + jnp.int32(i * _L)
            out_v[pl.ds(i * _L, _L)] = jnp.where(idxs == first, val, zero)

        pltpu.sync_copy(out_v, out_hbm)


_router = functools.partial(
    pl.kernel,
    mesh=plsc.VectorSubcoreMesh(core_axis_name="c", subcore_axis_name="s"),
    out_type=jax.ShapeDtypeStruct((_N,), jnp.float32),
    scratch_types=[
        pltpu.VMEM((_N,), jnp.float32),
        pltpu.VMEM((_N,), jnp.float32),
    ],
    compiler_params=pltpu.CompilerParams(needs_layout_passes=False),
)(_router_body)


def kernel(weight):
    return _router(weight)


# speedup vs baseline: 1.0740x; 1.0740x over previous
import functools

import jax
import jax.numpy as jnp
from jax import lax
from jax.experimental import pallas as pl
from jax.experimental.pallas import tpu as pltpu
from jax.experimental.pallas import tpu_sc as plsc

_N = 64
_INV_TEMP = 1.0 / (1.0 + 1e-10)
_EPS = 1e-10
_L = 16
_NV = _N // _L


def _router_body(w_hbm, out_hbm, w_v, out_v):
    pltpu.sync_copy(w_hbm, w_v)
    vs = [w_v[pl.ds(i * _L, _L)] for i in range(_NV)]

    m = jnp.max(vs[0])
    for v in vs[1:]:
        m = jnp.maximum(m, jnp.max(v))

    s = jnp.float32(0.0)
    for v in vs:
        s = s + jnp.sum(jnp.exp((v - m) * jnp.float32(_INV_TEMP)))

    iota = lax.iota(jnp.int32, _L)
    first = jnp.int32(_N)
    for i, v in enumerate(vs):
        cand = jnp.where(v == m, iota + jnp.int32(i * _L), jnp.int32(_N))
        first = jnp.minimum(first, jnp.min(cand))

    ones = jnp.ones((_L,), jnp.float32)
    denom = jnp.full((_L,), 1.0, jnp.float32) + jnp.float32(_EPS) * s
    val = ones / denom
    zero = jnp.zeros((_L,), jnp.float32)
    for i in range(_NV):
        idxs = iota + jnp.int32(i * _L)
        out_v[pl.ds(i * _L, _L)] = jnp.where(idxs == first, val, zero)

    pltpu.sync_copy(out_v, out_hbm)


_router = functools.partial(
    pl.kernel,
    mesh=plsc.VectorSubcoreMesh(
        core_axis_name="c", subcore_axis_name="s", num_cores=1, num_subcores=1
    ),
    out_type=jax.ShapeDtypeStruct((_N,), jnp.float32),
    scratch_types=[
        pltpu.VMEM((_N,), jnp.float32),
        pltpu.VMEM((_N,), jnp.float32),
    ],
    compiler_params=pltpu.CompilerParams(needs_layout_passes=False),
)(_router_body)


def kernel(weight):
    return _router(weight)


# speedup vs baseline: 1.0751x; 1.0010x over previous
import functools

import jax
import jax.numpy as jnp
from jax import lax
from jax.experimental import pallas as pl
from jax.experimental.pallas import tpu as pltpu
from jax.experimental.pallas import tpu_sc as plsc

_N = 64
_INV_TEMP = 1.0 / (1.0 + 1e-10)
_EPS = 1e-10
_L = 16
_NV = _N // _L


def _router_body(w_hbm, out_hbm, w_v, out_v):
    pltpu.sync_copy(w_hbm, w_v)
    vs = [w_v[pl.ds(i * _L, _L)] for i in range(_NV)]

    m = jnp.max(vs[0])
    for v in vs[1:]:
        m = jnp.maximum(m, jnp.max(v))

    s = jnp.float32(0.0)
    for v in vs:
        s = s + jnp.sum(jnp.exp((v - m) * jnp.float32(_INV_TEMP)))

    iota = lax.iota(jnp.int32, _L)
    first = jnp.int32(_N)
    for i, v in enumerate(vs):
        cand = jnp.where(v == m, iota + jnp.int32(i * _L), jnp.int32(_N))
        first = jnp.minimum(first, jnp.min(cand))

    ones = jnp.ones((_L,), jnp.float32)
    denom = jnp.full((_L,), 1.0, jnp.float32) + jnp.float32(_EPS) * s
    val = ones / denom
    zero = jnp.zeros((_L,), jnp.float32)
    for i in range(_NV):
        idxs = iota + jnp.int32(i * _L)
        out_v[pl.ds(i * _L, _L)] = jnp.where(idxs == first, val, zero)

    pltpu.sync_copy(out_v, out_hbm)


_router = functools.partial(
    pl.kernel,
    mesh=plsc.VectorSubcoreMesh(
        core_axis_name="c", subcore_axis_name="s", num_cores=1, num_subcores=1
    ),
    out_type=jax.ShapeDtypeStruct((_N,), jnp.float32),
    scratch_types=[
        pltpu.VMEM((_N,), jnp.float32),
        pltpu.VMEM((_N,), jnp.float32),
    ],
    compiler_params=pltpu.CompilerParams(
        needs_layout_passes=False, skip_device_barrier=True
    ),
)(_router_body)


def kernel(weight):
    return _router(weight)


# speedup vs baseline: 1.0835x; 1.0078x over previous
import functools

import jax
import jax.numpy as jnp
from jax import lax
from jax.experimental import pallas as pl
from jax.experimental.pallas import tpu as pltpu
from jax.experimental.pallas import tpu_sc as plsc

_N = 64
_INV_TEMP = 1.0 / (1.0 + 1e-10)
_EPS = 1e-10
_L = 16
_NV = _N // _L


def _router_body(w_hbm, out_hbm, w_v, out_v):
    pltpu.sync_copy(w_hbm, w_v)
    vs = [w_v[pl.ds(i * _L, _L)] for i in range(_NV)]

    mvec = jnp.maximum(jnp.maximum(vs[0], vs[1]), jnp.maximum(vs[2], vs[3]))
    m = jnp.max(mvec)

    evec = jnp.exp((vs[0] - m) * jnp.float32(_INV_TEMP))
    for v in vs[1:]:
        evec = evec + jnp.exp((v - m) * jnp.float32(_INV_TEMP))
    s = jnp.sum(evec)

    iota = lax.iota(jnp.int32, _L)
    cvec = jnp.full((_L,), _N, jnp.int32)
    for i, v in enumerate(vs):
        cvec = jnp.minimum(
            cvec, jnp.where(v == m, iota + jnp.int32(i * _L), jnp.int32(_N))
        )
    first = jnp.min(cvec)

    ones = jnp.ones((_L,), jnp.float32)
    denom = jnp.full((_L,), 1.0, jnp.float32) + jnp.float32(_EPS) * s
    val = ones / denom
    zero = jnp.zeros((_L,), jnp.float32)
    for i in range(_NV):
        idxs = iota + jnp.int32(i * _L)
        out_v[pl.ds(i * _L, _L)] = jnp.where(idxs == first, val, zero)

    pltpu.sync_copy(out_v, out_hbm)


_router = functools.partial(
    pl.kernel,
    mesh=plsc.VectorSubcoreMesh(
        core_axis_name="c", subcore_axis_name="s", num_cores=1, num_subcores=1
    ),
    out_type=jax.ShapeDtypeStruct((_N,), jnp.float32),
    scratch_types=[
        pltpu.VMEM((_N,), jnp.float32),
        pltpu.VMEM((_N,), jnp.float32),
    ],
    compiler_params=pltpu.CompilerParams(needs_layout_passes=False),
)(_router_body)


def kernel(weight):
    return _router(weight)
